# Initial kernel scaffold; baseline (speedup 1.0000x reference)
import jax
import jax.numpy as jnp
from jax.experimental import pallas as pl


def kernel(x_real, x_imag, edge_index, norm_real, norm_imag, W1, b1, W2, b2, Wc1, bc1, g1, be1, Wc2, bc2):
    raise NotImplementedError("write your pallas kernel here")


# speedup vs baseline: 3.4980x; 3.4980x over previous
import jax
import jax.numpy as jnp
from jax import lax
from jax.experimental import pallas as pl
from jax.experimental.pallas import tpu as pltpu
from jax.experimental.pallas import tpu_sc as plsc

N = 10000
E = 320000
D = 128
CW = 32
NCORE = 2
NSUB = 16
NW = NCORE * NSUB
ET = E // NW
K = 80
NCH = ET // K
NPAD = 10240
RPT = NPAD // NSUB
BN = 2000


def _scale_chunk(nbuf, gbuf, obuf):
  def group(q, inner):
    nr16 = nbuf[0, pl.ds(q * 16, 16)]
    ni16 = nbuf[1, pl.ds(q * 16, 16)]
    for j2 in range(16):
      nr_s = nr16[j2]
      ni_s = ni16[j2]
      j = q * 16 + j2
      for v in range(2):
        gr = gbuf[j, pl.ds(v * 16, 16)]
        gi = gbuf[j, pl.ds(32 + v * 16, 16)]
        obuf[j, pl.ds(v * 16, 16)] = nr_s * gr
        obuf[j, pl.ds(32 + v * 16, 16)] = ni_s * gi
        obuf[j, pl.ds(64 + v * 16, 16)] = nr_s * gi
        obuf[j, pl.ds(96 + v * 16, 16)] = ni_s * gr
    return inner
  lax.fori_loop(0, K // 16, group, 0)


def _sc_agg_body(t0, t1, t2, t3, e_hbm, n_hbm, out, ebuf0, ebuf1, nbuf0,
                 nbuf1, gbuf0, gbuf1, obuf, acc, gsem0, gsem1):
  cid = lax.axis_index("c")
  sid = lax.axis_index("s")
  wid = cid * NSUB + sid
  ebuf = (ebuf0, ebuf1)
  nbuf = (nbuf0, nbuf1)
  gbuf = (gbuf0, gbuf1)
  gsem = (gsem0, gsem1)

  for chunk_id, table in enumerate((t0, t1, t2, t3)):
    def zrow(r, carry):
      for v in range(8):
        obuf[r, pl.ds(v * 16, 16)] = jnp.zeros((16,), jnp.float32)
      return carry
    lax.fori_loop(0, K, zrow, 0)
    for z in range(RPT // K):
      pltpu.sync_copy(obuf, acc.at[pl.ds(sid * RPT + z * K, K)])
    plsc.subcore_barrier()

    pltpu.sync_copy(e_hbm.at[wid, 0], ebuf0)
    pltpu.sync_copy(n_hbm.at[wid, 0], nbuf0)
    pltpu.async_copy(table.at[ebuf0.at[0]], gbuf0, gsem0)

    def pair(c2, carry, table=table):
      for b in range(2):
        cc = 2 * c2 + b
        o = 1 - b
        pltpu.sync_copy(e_hbm.at[wid, cc + 1], ebuf[o])
        pltpu.sync_copy(n_hbm.at[wid, cc + 1], nbuf[o])
        pltpu.make_async_copy(table.at[ebuf[b].at[0]], gbuf[b], gsem[b]).wait()
        pltpu.async_copy(table.at[ebuf[o].at[0]], gbuf[o], gsem[o])
        _scale_chunk(nbuf[b], gbuf[b], obuf)
        pltpu.sync_copy(obuf, acc.at[ebuf[b].at[1]], add=True)
      return carry
    lax.fori_loop(0, (NCH - 1) // 2, pair, 0)

    pltpu.make_async_copy(table.at[ebuf0.at[0]], gbuf0, gsem0).wait()
    _scale_chunk(nbuf0, gbuf0, obuf)
    pltpu.sync_copy(obuf, acc.at[ebuf0.at[1]], add=True)

    plsc.subcore_barrier()
    pltpu.sync_copy(acc.at[pl.ds(sid * RPT, RPT)],
                    out.at[cid, chunk_id, pl.ds(sid * RPT, RPT)])
    plsc.subcore_barrier()


def _sc_agg(t0, t1, t2, t3, e_r, n_r):
  mesh = plsc.VectorSubcoreMesh(core_axis_name="c", subcore_axis_name="s",
                                num_cores=NCORE, num_subcores=NSUB)
  fn = pl.kernel(
      _sc_agg_body,
      out_type=jax.ShapeDtypeStruct((NCORE, 4, NPAD, D), jnp.float32),
      mesh=mesh,
      compiler_params=pltpu.CompilerParams(use_tc_tiling_on_sc=False),
      scratch_types=[
          pltpu.VMEM((2, K), jnp.int32),
          pltpu.VMEM((2, K), jnp.int32),
          pltpu.VMEM((2, K), jnp.float32),
          pltpu.VMEM((2, K), jnp.float32),
          pltpu.VMEM((K, 2 * CW), jnp.float32),
          pltpu.VMEM((K, 2 * CW), jnp.float32),
          pltpu.VMEM((K, D), jnp.float32),
          pltpu.MemorySpace.VMEM_SHARED((NPAD, D), jnp.float32),
          pltpu.SemaphoreType.DMA,
          pltpu.SemaphoreType.DMA,
      ],
  )
  return fn(t0, t1, t2, t3, e_r, n_r)


def _props(agg_ref):
  a = agg_ref[0] + agg_ref[1]
  ps = []
  for p in range(4):
    ps.append(jnp.concatenate(
        [a[c, :, p * CW:(p + 1) * CW] for c in range(4)], axis=1))
  return ps


def _layer(xr, xi, agg_ref, w0_ref, w1_ref, b_ref):
  p1, p2, p3, p4 = _props(agg_ref)
  w0 = w0_ref[...]
  w1 = w1_ref[...]
  b = b_ref[...]
  xrw0 = jnp.dot(xr, w0, preferred_element_type=jnp.float32)
  xiw0 = jnp.dot(xi, w0, preferred_element_type=jnp.float32)
  out_rr = xrw0 + jnp.dot(p1, w1, preferred_element_type=jnp.float32)
  out_ii = xiw0 + jnp.dot(p2, w1, preferred_element_type=jnp.float32)
  out_ir = xiw0 + jnp.dot(p3, w1, preferred_element_type=jnp.float32)
  out_ri = xrw0 + jnp.dot(p4, w1, preferred_element_type=jnp.float32)
  r = out_rr - out_ii + b
  i = out_ir + out_ri + b
  mask = (r >= 0.0).astype(r.dtype)
  return mask * r, mask * i


def _tc_layer1_body(xr_ref, xi_ref, agg_ref, w0_ref, w1_ref, b_ref, y_ref):
  r, i = _layer(xr_ref[...], xi_ref[...], agg_ref, w0_ref, w1_ref, b_ref)
  for c in range(4):
    y_ref[c] = jnp.concatenate(
        [r[:, c * CW:(c + 1) * CW], i[:, c * CW:(c + 1) * CW]], axis=1)


def _tc_layer2_body(y_ref, agg_ref, w0_ref, w1_ref, b_ref,
                    wc1_ref, bc1_ref, g1_ref, be1_ref, wc2_ref, bc2_ref,
                    out_ref):
  r1 = jnp.concatenate([y_ref[c, :, :CW] for c in range(4)], axis=1)
  i1 = jnp.concatenate([y_ref[c, :, CW:] for c in range(4)], axis=1)
  r, i = _layer(r1, i1, agg_ref, w0_ref, w1_ref, b_ref)
  h = jnp.concatenate([r, i], axis=1)
  h = jnp.dot(h, wc1_ref[...], preferred_element_type=jnp.float32)
  h = h + bc1_ref[...]
  mu = jnp.mean(h, axis=-1, keepdims=True)
  var = jnp.mean((h - mu) * (h - mu), axis=-1, keepdims=True)
  h = (h - mu) / jnp.sqrt(var + 1e-5) * g1_ref[...] + be1_ref[...]
  h = jnp.maximum(h, 0.0)
  logits = jnp.dot(h, wc2_ref[...], preferred_element_type=jnp.float32)
  logits = logits + bc2_ref[...]
  m = jnp.max(logits, axis=-1, keepdims=True)
  shifted = logits - m
  out_ref[...] = shifted - jnp.log(
      jnp.sum(jnp.exp(shifted), axis=-1, keepdims=True))


def _row_spec(shape):
  nd = len(shape)
  return pl.BlockSpec(shape, lambda i: (0,) * (nd - 2) + (i, 0))


def _full_spec(shape):
  nd = len(shape)
  return pl.BlockSpec(shape, lambda i: (0,) * nd)


def _tc_layer1(xr, xi, agg, w0, w1, b):
  return pl.pallas_call(
      _tc_layer1_body,
      grid=(N // BN,),
      in_specs=[
          _row_spec((BN, D)),
          _row_spec((BN, D)),
          _row_spec((NCORE, 4, BN, D)),
          _full_spec((D, D)),
          _full_spec((D, D)),
          _full_spec((1, D)),
      ],
      out_specs=[_row_spec((4, BN, 2 * CW))],
      out_shape=[jax.ShapeDtypeStruct((4, N, 2 * CW), jnp.float32)],
  )(xr, xi, agg, w0, w1, b)[0]


def _tc_layer2(y, agg, w0, w1, b, wc1, bc1, g1, be1, wc2, bc2):
  return pl.pallas_call(
      _tc_layer2_body,
      grid=(N // BN,),
      in_specs=[
          _row_spec((4, BN, 2 * CW)),
          _row_spec((NCORE, 4, BN, D)),
          _full_spec((D, D)),
          _full_spec((D, D)),
          _full_spec((1, D)),
          _full_spec((2 * D, 64)),
          _full_spec((1, 64)),
          _full_spec((1, 64)),
          _full_spec((1, 64)),
          _full_spec((64, 10)),
          _full_spec((1, 10)),
      ],
      out_specs=[_row_spec((BN, 10))],
      out_shape=[jax.ShapeDtypeStruct((N, 10), jnp.float32)],
  )(y, agg, w0, w1, b, wc1, bc1, g1, be1, wc2, bc2)[0]


def kernel(x_real, x_imag, edge_index, norm_real, norm_imag,
           W1, b1, W2, b2, Wc1, bc1, g1, be1, Wc2, bc2):
  src_r = edge_index[0].reshape(NW, NCH, K)
  dst_r = edge_index[1].reshape(NW, NCH, K)
  nr_r = norm_real.reshape(NW, NCH, K)
  ni_r = norm_imag.reshape(NW, NCH, K)
  e_r = jnp.stack([dst_r, src_r], axis=2)
  n_r = jnp.stack([nr_r, ni_r], axis=2)

  t = [jnp.concatenate([x_real[:, c * CW:(c + 1) * CW],
                        x_imag[:, c * CW:(c + 1) * CW]], axis=1)
       for c in range(4)]
  agg1 = _sc_agg(t[0], t[1], t[2], t[3], e_r, n_r)
  y = _tc_layer1(x_real, x_imag, agg1, W1[0], W1[1], b1.reshape(1, D))

  agg2 = _sc_agg(y[0], y[1], y[2], y[3], e_r, n_r)
  return _tc_layer2(y, agg2, W2[0], W2[1], b2.reshape(1, D),
                    Wc1, bc1.reshape(1, 64), g1.reshape(1, 64),
                    be1.reshape(1, 64), Wc2, bc2.reshape(1, 10))


# speedup vs baseline: 4.1710x; 1.1924x over previous
import jax
import jax.numpy as jnp
from jax import lax
from jax.experimental import pallas as pl
from jax.experimental.pallas import tpu as pltpu
from jax.experimental.pallas import tpu_sc as plsc

N = 10000
E = 320000
D = 128
CW = 32
NCORE = 2
NSUB = 16
NW = NCORE * NSUB
ET = E // NW
K = 80
NCH = ET // K
NPAD = 10240
RPT = NPAD // NSUB
BN = 2000


def _scale_chunk(nbuf, gbuf, obuf):
  def group(q, inner):
    nr16 = nbuf[0, pl.ds(q * 16, 16)]
    ni16 = nbuf[1, pl.ds(q * 16, 16)]
    for j2 in range(16):
      nr_s = nr16[j2]
      ni_s = ni16[j2]
      j = q * 16 + j2
      for v in range(2):
        gr = gbuf[j, pl.ds(v * 16, 16)]
        gi = gbuf[j, pl.ds(32 + v * 16, 16)]
        obuf[j, pl.ds(v * 16, 16)] = nr_s * gr
        obuf[j, pl.ds(32 + v * 16, 16)] = ni_s * gi
        obuf[j, pl.ds(64 + v * 16, 16)] = nr_s * gi
        obuf[j, pl.ds(96 + v * 16, 16)] = ni_s * gr
    return inner
  lax.fori_loop(0, K // 16, group, 0)


def _sc_agg_body(t0, t1, t2, t3, e_hbm, n_hbm, out, ebuf0, ebuf1, nbuf0,
                 nbuf1, gbuf0, gbuf1, obuf, acc, gsem0, gsem1, fsem0, fsem1):
  cid = lax.axis_index("c")
  sid = lax.axis_index("s")
  wid = cid * NSUB + sid
  ebuf = (ebuf0, ebuf1)
  nbuf = (nbuf0, nbuf1)
  gbuf = (gbuf0, gbuf1)
  gsem = (gsem0, gsem1)
  fsem = (fsem0, fsem1)

  for chunk_id, table in enumerate((t0, t1, t2, t3)):
    def zrow(r, carry):
      for v in range(8):
        obuf[r, pl.ds(v * 16, 16)] = jnp.zeros((16,), jnp.float32)
      return carry
    lax.fori_loop(0, K, zrow, 0)
    for z in range(RPT // K):
      pltpu.sync_copy(obuf, acc.at[pl.ds(sid * RPT + z * K, K)])
    plsc.subcore_barrier()

    pltpu.sync_copy(e_hbm.at[wid, 0], ebuf0)
    pltpu.sync_copy(n_hbm.at[wid, 0], nbuf0)
    pltpu.async_copy(table.at[ebuf0.at[0]], gbuf0, gsem0)
    pltpu.async_copy(e_hbm.at[wid, 1], ebuf1, fsem1)
    pltpu.async_copy(n_hbm.at[wid, 1], nbuf1, fsem1)

    def pair(c2, carry, table=table):
      for b in range(2):
        cc = 2 * c2 + b
        o = 1 - b
        pltpu.make_async_copy(e_hbm.at[wid, cc + 1], ebuf[o], fsem[o]).wait()
        pltpu.make_async_copy(n_hbm.at[wid, cc + 1], nbuf[o], fsem[o]).wait()
        pltpu.make_async_copy(table.at[ebuf[b].at[0]], gbuf[b], gsem[b]).wait()
        pltpu.async_copy(table.at[ebuf[o].at[0]], gbuf[o], gsem[o])
        _scale_chunk(nbuf[b], gbuf[b], obuf)
        pltpu.sync_copy(obuf, acc.at[ebuf[b].at[1]], add=True)
        pltpu.async_copy(e_hbm.at[wid, cc + 2], ebuf[b], fsem[b])
        pltpu.async_copy(n_hbm.at[wid, cc + 2], nbuf[b], fsem[b])
      return carry
    lax.fori_loop(0, (NCH - 1) // 2, pair, 0)

    pltpu.make_async_copy(table.at[ebuf0.at[0]], gbuf0, gsem0).wait()
    _scale_chunk(nbuf0, gbuf0, obuf)
    pltpu.sync_copy(obuf, acc.at[ebuf0.at[1]], add=True)
    pltpu.make_async_copy(e_hbm.at[wid, NCH], ebuf1, fsem1).wait()
    pltpu.make_async_copy(n_hbm.at[wid, NCH], nbuf1, fsem1).wait()

    plsc.subcore_barrier()
    pltpu.sync_copy(acc.at[pl.ds(sid * RPT, RPT)],
                    out.at[cid, chunk_id, pl.ds(sid * RPT, RPT)])
    plsc.subcore_barrier()


def _sc_agg(t0, t1, t2, t3, e_r, n_r):
  mesh = plsc.VectorSubcoreMesh(core_axis_name="c", subcore_axis_name="s",
                                num_cores=NCORE, num_subcores=NSUB)
  fn = pl.kernel(
      _sc_agg_body,
      out_type=jax.ShapeDtypeStruct((NCORE, 4, NPAD, D), jnp.float32),
      mesh=mesh,
      compiler_params=pltpu.CompilerParams(use_tc_tiling_on_sc=False),
      scratch_types=[
          pltpu.VMEM((2, K), jnp.int32),
          pltpu.VMEM((2, K), jnp.int32),
          pltpu.VMEM((2, K), jnp.float32),
          pltpu.VMEM((2, K), jnp.float32),
          pltpu.VMEM((K, 2 * CW), jnp.float32),
          pltpu.VMEM((K, 2 * CW), jnp.float32),
          pltpu.VMEM((K, D), jnp.float32),
          pltpu.MemorySpace.VMEM_SHARED((NPAD, D), jnp.float32),
          pltpu.SemaphoreType.DMA,
          pltpu.SemaphoreType.DMA,
          pltpu.SemaphoreType.DMA,
          pltpu.SemaphoreType.DMA,
      ],
  )
  return fn(t0, t1, t2, t3, e_r, n_r)


def _props(agg_ref):
  a = agg_ref[0] + agg_ref[1]
  ps = []
  for p in range(4):
    ps.append(jnp.concatenate(
        [a[c, :, p * CW:(p + 1) * CW] for c in range(4)], axis=1))
  return ps


def _layer(xr, xi, agg_ref, w0_ref, w1_ref, b_ref):
  p1, p2, p3, p4 = _props(agg_ref)
  w0 = w0_ref[...]
  w1 = w1_ref[...]
  b = b_ref[...]
  xrw0 = jnp.dot(xr, w0, preferred_element_type=jnp.float32)
  xiw0 = jnp.dot(xi, w0, preferred_element_type=jnp.float32)
  out_rr = xrw0 + jnp.dot(p1, w1, preferred_element_type=jnp.float32)
  out_ii = xiw0 + jnp.dot(p2, w1, preferred_element_type=jnp.float32)
  out_ir = xiw0 + jnp.dot(p3, w1, preferred_element_type=jnp.float32)
  out_ri = xrw0 + jnp.dot(p4, w1, preferred_element_type=jnp.float32)
  r = out_rr - out_ii + b
  i = out_ir + out_ri + b
  mask = (r >= 0.0).astype(r.dtype)
  return mask * r, mask * i


def _tc_layer1_body(xr_ref, xi_ref, agg_ref, w0_ref, w1_ref, b_ref, y_ref):
  r, i = _layer(xr_ref[...], xi_ref[...], agg_ref, w0_ref, w1_ref, b_ref)
  for c in range(4):
    y_ref[c] = jnp.concatenate(
        [r[:, c * CW:(c + 1) * CW], i[:, c * CW:(c + 1) * CW]], axis=1)


def _tc_layer2_body(y_ref, agg_ref, w0_ref, w1_ref, b_ref,
                    wc1_ref, bc1_ref, g1_ref, be1_ref, wc2_ref, bc2_ref,
                    out_ref):
  r1 = jnp.concatenate([y_ref[c, :, :CW] for c in range(4)], axis=1)
  i1 = jnp.concatenate([y_ref[c, :, CW:] for c in range(4)], axis=1)
  r, i = _layer(r1, i1, agg_ref, w0_ref, w1_ref, b_ref)
  h = jnp.concatenate([r, i], axis=1)
  h = jnp.dot(h, wc1_ref[...], preferred_element_type=jnp.float32)
  h = h + bc1_ref[...]
  mu = jnp.mean(h, axis=-1, keepdims=True)
  var = jnp.mean((h - mu) * (h - mu), axis=-1, keepdims=True)
  h = (h - mu) / jnp.sqrt(var + 1e-5) * g1_ref[...] + be1_ref[...]
  h = jnp.maximum(h, 0.0)
  logits = jnp.dot(h, wc2_ref[...], preferred_element_type=jnp.float32)
  logits = logits + bc2_ref[...]
  m = jnp.max(logits, axis=-1, keepdims=True)
  shifted = logits - m
  out_ref[...] = shifted - jnp.log(
      jnp.sum(jnp.exp(shifted), axis=-1, keepdims=True))


def _row_spec(shape):
  nd = len(shape)
  return pl.BlockSpec(shape, lambda i: (0,) * (nd - 2) + (i, 0))


def _full_spec(shape):
  nd = len(shape)
  return pl.BlockSpec(shape, lambda i: (0,) * nd)


def _tc_layer1(xr, xi, agg, w0, w1, b):
  return pl.pallas_call(
      _tc_layer1_body,
      grid=(N // BN,),
      in_specs=[
          _row_spec((BN, D)),
          _row_spec((BN, D)),
          _row_spec((NCORE, 4, BN, D)),
          _full_spec((D, D)),
          _full_spec((D, D)),
          _full_spec((1, D)),
      ],
      out_specs=[_row_spec((4, BN, 2 * CW))],
      out_shape=[jax.ShapeDtypeStruct((4, N, 2 * CW), jnp.float32)],
  )(xr, xi, agg, w0, w1, b)[0]


def _tc_layer2(y, agg, w0, w1, b, wc1, bc1, g1, be1, wc2, bc2):
  return pl.pallas_call(
      _tc_layer2_body,
      grid=(N // BN,),
      in_specs=[
          _row_spec((4, BN, 2 * CW)),
          _row_spec((NCORE, 4, BN, D)),
          _full_spec((D, D)),
          _full_spec((D, D)),
          _full_spec((1, D)),
          _full_spec((2 * D, 64)),
          _full_spec((1, 64)),
          _full_spec((1, 64)),
          _full_spec((1, 64)),
          _full_spec((64, 10)),
          _full_spec((1, 10)),
      ],
      out_specs=[_row_spec((BN, 10))],
      out_shape=[jax.ShapeDtypeStruct((N, 10), jnp.float32)],
  )(y, agg, w0, w1, b, wc1, bc1, g1, be1, wc2, bc2)[0]


def kernel(x_real, x_imag, edge_index, norm_real, norm_imag,
           W1, b1, W2, b2, Wc1, bc1, g1, be1, Wc2, bc2):
  src_r = edge_index[0].reshape(NW, NCH, K)
  dst_r = edge_index[1].reshape(NW, NCH, K)
  nr_r = norm_real.reshape(NW, NCH, K)
  ni_r = norm_imag.reshape(NW, NCH, K)
  e_r = jnp.concatenate(
      [jnp.stack([dst_r, src_r], axis=2),
       jnp.zeros((NW, 1, 2, K), jnp.int32)], axis=1)
  n_r = jnp.concatenate(
      [jnp.stack([nr_r, ni_r], axis=2),
       jnp.zeros((NW, 1, 2, K), jnp.float32)], axis=1)

  t = [jnp.concatenate([x_real[:, c * CW:(c + 1) * CW],
                        x_imag[:, c * CW:(c + 1) * CW]], axis=1)
       for c in range(4)]
  agg1 = _sc_agg(t[0], t[1], t[2], t[3], e_r, n_r)
  y = _tc_layer1(x_real, x_imag, agg1, W1[0], W1[1], b1.reshape(1, D))

  agg2 = _sc_agg(y[0], y[1], y[2], y[3], e_r, n_r)
  return _tc_layer2(y, agg2, W2[0], W2[1], b2.reshape(1, D),
                    Wc1, bc1.reshape(1, 64), g1.reshape(1, 64),
                    be1.reshape(1, 64), Wc2, bc2.reshape(1, 10))


# speedup vs baseline: 5.1013x; 1.2230x over previous
import jax
import jax.numpy as jnp
from jax import lax
from jax.experimental import pallas as pl
from jax.experimental.pallas import tpu as pltpu
from jax.experimental.pallas import tpu_sc as plsc

N = 10000
E = 320000
D = 128
CW = 32
NCORE = 2
NSUB = 16
NW = NCORE * NSUB
ET = E // NW
K = 80
NCH = ET // K
NPAD = 10240
RPT = NPAD // NSUB
BN = 2000


def _scale_chunk(nbuf, gbuf, obuf):
  def group(q, inner):
    nr16 = nbuf[0, pl.ds(q * 16, 16)]
    ni16 = nbuf[1, pl.ds(q * 16, 16)]
    for j2 in range(16):
      nr_s = nr16[j2]
      ni_s = ni16[j2]
      j = q * 16 + j2
      for v in range(2):
        gr = gbuf[j, pl.ds(v * 16, 16)]
        gi = gbuf[j, pl.ds(32 + v * 16, 16)]
        obuf[j, pl.ds(v * 16, 16)] = nr_s * gr
        obuf[j, pl.ds(32 + v * 16, 16)] = ni_s * gi
        obuf[j, pl.ds(64 + v * 16, 16)] = nr_s * gi
        obuf[j, pl.ds(96 + v * 16, 16)] = ni_s * gr
    return inner
  lax.fori_loop(0, K // 16, group, 0)


def _sc_agg_body(tables, e_hbm, n_hbm, out,
                 ebuf0, ebuf1, ebuf2, ebuf3, nbuf0, nbuf1, nbuf2, nbuf3,
                 gbuf0, gbuf1, obuf0, obuf1, acc,
                 gsem0, gsem1, ssem0, ssem1, fsem0, fsem1, fsem2, fsem3):
  cid = lax.axis_index("c")
  sid = lax.axis_index("s")
  wid = cid * NSUB + sid
  ebuf = (ebuf0, ebuf1, ebuf2, ebuf3)
  nbuf = (nbuf0, nbuf1, nbuf2, nbuf3)
  gbuf = (gbuf0, gbuf1)
  obuf = (obuf0, obuf1)
  gsem = (gsem0, gsem1)
  ssem = (ssem0, ssem1)
  fsem = (fsem0, fsem1, fsem2, fsem3)

  def pass_body(p, pass_carry):
    table = tables.at[p]
    def zrow(r, carry):
      for v in range(8):
        obuf0[r, pl.ds(v * 16, 16)] = jnp.zeros((16,), jnp.float32)
      return carry
    lax.fori_loop(0, K, zrow, 0)
    for z in range(RPT // K):
      pltpu.sync_copy(obuf0, acc.at[pl.ds(sid * RPT + z * K, K)])
    plsc.subcore_barrier()

    def chunk(cc, b2, b4, table, ssem_wait=True, wait_fetch=True,
              gather_next=True, fetch_next=True):
      o2 = 1 - b2
      nb4 = (b4 + 1) % 4
      fb4 = (b4 + 2) % 4
      if ssem_wait:
        pltpu.make_async_copy(obuf[b2], acc.at[ebuf[b4].at[1]],
                              ssem[b2]).wait()
      if gather_next and wait_fetch:
        pltpu.make_async_copy(e_hbm.at[wid, cc + 1], ebuf[nb4],
                              fsem[nb4]).wait()
        pltpu.make_async_copy(n_hbm.at[wid, cc + 1], nbuf[nb4],
                              fsem[nb4]).wait()
      pltpu.make_async_copy(table.at[ebuf[b4].at[0]], gbuf[b2],
                            gsem[b2]).wait()
      if gather_next:
        pltpu.async_copy(table.at[ebuf[nb4].at[0]], gbuf[o2], gsem[o2])
      _scale_chunk(nbuf[b4], gbuf[b2], obuf[b2])
      pltpu.async_copy(obuf[b2], acc.at[ebuf[b4].at[1]], ssem[b2], add=True)
      if fetch_next:
        pltpu.async_copy(e_hbm.at[wid, cc + 2], ebuf[fb4], fsem[fb4])
        pltpu.async_copy(n_hbm.at[wid, cc + 2], nbuf[fb4], fsem[fb4])

    pltpu.sync_copy(e_hbm.at[wid, 0], ebuf0)
    pltpu.sync_copy(n_hbm.at[wid, 0], nbuf0)
    pltpu.sync_copy(e_hbm.at[wid, 1], ebuf1)
    pltpu.sync_copy(n_hbm.at[wid, 1], nbuf1)
    pltpu.async_copy(table.at[ebuf0.at[0]], gbuf0, gsem0)
    chunk(0, 0, 0, table, ssem_wait=False, wait_fetch=False)
    chunk(1, 1, 1, table, ssem_wait=False)

    def quad(c4, carry, table=table):
      for u in range(4):
        cc = 4 * c4 + 2 + u
        chunk(cc, (2 + u) % 2, (2 + u) % 4, table)
      return carry
    lax.fori_loop(0, (NCH - 5) // 4, quad, 0)

    chunk(NCH - 3, 0, (NCH - 3) % 4, table)
    chunk(NCH - 2, 1, (NCH - 2) % 4, table, fetch_next=False)
    chunk(NCH - 1, 0, (NCH - 1) % 4, table, gather_next=False,
          fetch_next=False)
    pltpu.make_async_copy(obuf1, acc.at[ebuf[(NCH - 2) % 4].at[1]],
                          ssem1).wait()
    pltpu.make_async_copy(obuf0, acc.at[ebuf[(NCH - 1) % 4].at[1]],
                          ssem0).wait()

    plsc.subcore_barrier()
    pltpu.sync_copy(acc.at[pl.ds(sid * RPT, RPT)],
                    out.at[cid, p, pl.ds(sid * RPT, RPT)])
    plsc.subcore_barrier()
    return pass_carry
  lax.fori_loop(0, 4, pass_body, 0)


def _sc_agg(tables, e_r, n_r):
  mesh = plsc.VectorSubcoreMesh(core_axis_name="c", subcore_axis_name="s",
                                num_cores=NCORE, num_subcores=NSUB)
  fn = pl.kernel(
      _sc_agg_body,
      out_type=jax.ShapeDtypeStruct((NCORE, 4, NPAD, D), jnp.float32),
      mesh=mesh,
      compiler_params=pltpu.CompilerParams(use_tc_tiling_on_sc=False),
      scratch_types=(
          [pltpu.VMEM((2, K), jnp.int32)] * 4
          + [pltpu.VMEM((2, K), jnp.float32)] * 4
          + [pltpu.VMEM((K, 2 * CW), jnp.float32)] * 2
          + [pltpu.VMEM((K, D), jnp.float32)] * 2
          + [pltpu.MemorySpace.VMEM_SHARED((NPAD, D), jnp.float32)]
          + [pltpu.SemaphoreType.DMA] * 8
      ),
  )
  return fn(tables, e_r, n_r)


def _props(agg_ref):
  a = agg_ref[0] + agg_ref[1]
  ps = []
  for p in range(4):
    ps.append(jnp.concatenate(
        [a[c, :, p * CW:(p + 1) * CW] for c in range(4)], axis=1))
  return ps


def _layer(xr, xi, agg_ref, w0_ref, w1_ref, b_ref):
  p1, p2, p3, p4 = _props(agg_ref)
  w0 = w0_ref[...]
  w1 = w1_ref[...]
  b = b_ref[...]
  xrw0 = jnp.dot(xr, w0, preferred_element_type=jnp.float32)
  xiw0 = jnp.dot(xi, w0, preferred_element_type=jnp.float32)
  out_rr = xrw0 + jnp.dot(p1, w1, preferred_element_type=jnp.float32)
  out_ii = xiw0 + jnp.dot(p2, w1, preferred_element_type=jnp.float32)
  out_ir = xiw0 + jnp.dot(p3, w1, preferred_element_type=jnp.float32)
  out_ri = xrw0 + jnp.dot(p4, w1, preferred_element_type=jnp.float32)
  r = out_rr - out_ii + b
  i = out_ir + out_ri + b
  mask = (r >= 0.0).astype(r.dtype)
  return mask * r, mask * i


def _tc_layer1_body(xr_ref, xi_ref, agg_ref, w0_ref, w1_ref, b_ref, y_ref):
  r, i = _layer(xr_ref[...], xi_ref[...], agg_ref, w0_ref, w1_ref, b_ref)
  for c in range(4):
    y_ref[c] = jnp.concatenate(
        [r[:, c * CW:(c + 1) * CW], i[:, c * CW:(c + 1) * CW]], axis=1)


def _tc_layer2_body(y_ref, agg_ref, w0_ref, w1_ref, b_ref,
                    wc1_ref, bc1_ref, g1_ref, be1_ref, wc2_ref, bc2_ref,
                    out_ref):
  r1 = jnp.concatenate([y_ref[c, :, :CW] for c in range(4)], axis=1)
  i1 = jnp.concatenate([y_ref[c, :, CW:] for c in range(4)], axis=1)
  r, i = _layer(r1, i1, agg_ref, w0_ref, w1_ref, b_ref)
  h = jnp.concatenate([r, i], axis=1)
  h = jnp.dot(h, wc1_ref[...], preferred_element_type=jnp.float32)
  h = h + bc1_ref[...]
  mu = jnp.mean(h, axis=-1, keepdims=True)
  var = jnp.mean((h - mu) * (h - mu), axis=-1, keepdims=True)
  h = (h - mu) / jnp.sqrt(var + 1e-5) * g1_ref[...] + be1_ref[...]
  h = jnp.maximum(h, 0.0)
  logits = jnp.dot(h, wc2_ref[...], preferred_element_type=jnp.float32)
  logits = logits + bc2_ref[...]
  m = jnp.max(logits, axis=-1, keepdims=True)
  shifted = logits - m
  out_ref[...] = shifted - jnp.log(
      jnp.sum(jnp.exp(shifted), axis=-1, keepdims=True))


def _row_spec(shape):
  nd = len(shape)
  return pl.BlockSpec(shape, lambda i: (0,) * (nd - 2) + (i, 0))


def _full_spec(shape):
  nd = len(shape)
  return pl.BlockSpec(shape, lambda i: (0,) * nd)


def _tc_layer1(xr, xi, agg, w0, w1, b):
  return pl.pallas_call(
      _tc_layer1_body,
      grid=(N // BN,),
      in_specs=[
          _row_spec((BN, D)),
          _row_spec((BN, D)),
          _row_spec((NCORE, 4, BN, D)),
          _full_spec((D, D)),
          _full_spec((D, D)),
          _full_spec((1, D)),
      ],
      out_specs=[_row_spec((4, BN, 2 * CW))],
      out_shape=[jax.ShapeDtypeStruct((4, N, 2 * CW), jnp.float32)],
  )(xr, xi, agg, w0, w1, b)[0]


def _tc_layer2(y, agg, w0, w1, b, wc1, bc1, g1, be1, wc2, bc2):
  return pl.pallas_call(
      _tc_layer2_body,
      grid=(N // BN,),
      in_specs=[
          _row_spec((4, BN, 2 * CW)),
          _row_spec((NCORE, 4, BN, D)),
          _full_spec((D, D)),
          _full_spec((D, D)),
          _full_spec((1, D)),
          _full_spec((2 * D, 64)),
          _full_spec((1, 64)),
          _full_spec((1, 64)),
          _full_spec((1, 64)),
          _full_spec((64, 10)),
          _full_spec((1, 10)),
      ],
      out_specs=[_row_spec((BN, 10))],
      out_shape=[jax.ShapeDtypeStruct((N, 10), jnp.float32)],
  )(y, agg, w0, w1, b, wc1, bc1, g1, be1, wc2, bc2)[0]


def kernel(x_real, x_imag, edge_index, norm_real, norm_imag,
           W1, b1, W2, b2, Wc1, bc1, g1, be1, Wc2, bc2):
  src_r = edge_index[0].reshape(NW, NCH, K)
  dst_r = edge_index[1].reshape(NW, NCH, K)
  nr_r = norm_real.reshape(NW, NCH, K)
  ni_r = norm_imag.reshape(NW, NCH, K)
  e_r = jnp.concatenate(
      [jnp.stack([dst_r, src_r], axis=2),
       jnp.zeros((NW, 1, 2, K), jnp.int32)], axis=1)
  n_r = jnp.concatenate(
      [jnp.stack([nr_r, ni_r], axis=2),
       jnp.zeros((NW, 1, 2, K), jnp.float32)], axis=1)

  t = [jnp.concatenate([x_real[:, c * CW:(c + 1) * CW],
                        x_imag[:, c * CW:(c + 1) * CW]], axis=1)
       for c in range(4)]
  agg1 = _sc_agg(jnp.stack(t, axis=0), e_r, n_r)
  y = _tc_layer1(x_real, x_imag, agg1, W1[0], W1[1], b1.reshape(1, D))

  agg2 = _sc_agg(y, e_r, n_r)
  return _tc_layer2(y, agg2, W2[0], W2[1], b2.reshape(1, D),
                    Wc1, bc1.reshape(1, 64), g1.reshape(1, 64),
                    be1.reshape(1, 64), Wc2, bc2.reshape(1, 10))
